# baseline (speedup 1.0000x reference)
import functools

import jax
import jax.numpy as jnp
from jax import lax
from jax.experimental import pallas as pl
from jax.experimental.pallas import tpu as pltpu
from jax.experimental.pallas import tpu_sc as plsc

B, T, D = 4, 2048, 768
VOCAB = 50257
NC, NS, L = 2, 16, 16
NW = NC * NS
TPW = T // NW
CS = 8
NGRP = TPW // CS
NBUF = 3

_mesh = plsc.VectorSubcoreMesh(
    core_axis_name="c", subcore_axis_name="s", num_cores=NC, num_subcores=NS
)


@functools.partial(
    pl.kernel,
    out_type=jax.ShapeDtypeStruct((B, T, D), jnp.float32),
    mesh=_mesh,
    scratch_types=[
        pltpu.VMEM((NGRP, B * CS), jnp.int32),
        [pltpu.VMEM((CS, D), jnp.float32) for _ in range(NBUF)],
        [pltpu.VMEM((B * CS, D), jnp.float32) for _ in range(NBUF)],
        pltpu.SemaphoreType.DMA,
        [pltpu.SemaphoreType.DMA for _ in range(NBUF)],
        [pltpu.SemaphoreType.DMA for _ in range(NBUF)],
        [pltpu.SemaphoreType.DMA for _ in range(NBUF)],
    ],
)
def _emb_lookup(ids_hbm, wte_hbm, wpe_hbm, out_hbm,
                idx_v, wpe_v, rows_v, isem, wsems, gsems, ssems):
    sid = lax.axis_index("s")
    wid = sid * NC + lax.axis_index("c")
    t0 = wid * TPW

    idx_descs = [
        pltpu.async_copy(
            ids_hbm.at[b, pl.ds(t0 + g * CS, CS)],
            idx_v.at[g, pl.ds(b * CS, CS)],
            isem,
        )
        for g in range(NGRP)
        for b in range(B)
    ]
    for d in idx_descs:
        d.wait()

    def start_gather(g):
        p = g % NBUF
        return pltpu.async_copy(
            wte_hbm.at[idx_v.at[g]], rows_v[p], gsems[p]
        )

    def start_wpe(g):
        p = g % NBUF
        return pltpu.async_copy(
            wpe_hbm.at[pl.ds(t0 + g * CS, CS), :], wpe_v[p], wsems[p]
        )

    def start_stores(g):
        p = g % NBUF
        return [
            pltpu.async_copy(
                rows_v[p].at[pl.ds(b * CS, CS)],
                out_hbm.at[b, pl.ds(t0 + g * CS, CS), :],
                ssems[p],
            )
            for b in range(B)
        ]

    def add_wpe(g):
        p = g % NBUF
        buf = rows_v[p]
        wp = wpe_v[p]

        @plsc.parallel_loop(0, CS, unroll=1)
        def _(i):
            for j in range(D // L):
                sl = pl.ds(j * L, L)
                x = wp[i, sl]
                for b in range(B):
                    plsc.addupdate(buf.at[b * CS + i, sl], x)

    g_descs = {}
    w_descs = {}
    s_descs = {}
    for g in range(NBUF):
        g_descs[g] = start_gather(g)
        w_descs[g] = start_wpe(g)
    for g in range(NGRP):
        j = g + NBUF - 1
        if 1 <= g and j < NGRP:
            for d in s_descs[g - 1]:
                d.wait()
            g_descs[j] = start_gather(j)
            w_descs[j] = start_wpe(j)
        g_descs[g].wait()
        w_descs[g].wait()
        add_wpe(g)
        s_descs[g] = start_stores(g)
    for g in range(max(0, NGRP - NBUF), NGRP):
        for d in s_descs[g]:
            d.wait()


def kernel(input_ids, wte, wpe):
    ids32 = input_ids.astype(jnp.int32)
    return _emb_lookup(ids32, wte, wpe)


# speedup vs baseline: 1.0273x; 1.0273x over previous
import functools

import jax
import jax.numpy as jnp
from jax import lax
from jax.experimental import pallas as pl
from jax.experimental.pallas import tpu as pltpu
from jax.experimental.pallas import tpu_sc as plsc

B, T, D = 4, 2048, 768
VOCAB = 50257
NC, NS, L = 2, 16, 16
NW = NC * NS
TPW = T // NW
CS = 16
NGRP = TPW // CS
NBUF = 2

_mesh = plsc.VectorSubcoreMesh(
    core_axis_name="c", subcore_axis_name="s", num_cores=NC, num_subcores=NS
)


@functools.partial(
    pl.kernel,
    out_type=jax.ShapeDtypeStruct((B, T, D), jnp.float32),
    mesh=_mesh,
    scratch_types=[
        pltpu.VMEM((NGRP, B * CS), jnp.int32),
        [pltpu.VMEM((CS, D), jnp.float32) for _ in range(NBUF)],
        [pltpu.VMEM((B * CS, D), jnp.float32) for _ in range(NBUF)],
        pltpu.SemaphoreType.DMA,
        [pltpu.SemaphoreType.DMA for _ in range(NBUF)],
        [pltpu.SemaphoreType.DMA for _ in range(NBUF)],
        [pltpu.SemaphoreType.DMA for _ in range(NBUF)],
    ],
)
def _emb_lookup(ids_hbm, wte_hbm, wpe_hbm, out_hbm,
                idx_v, wpe_v, rows_v, isem, wsems, gsems, ssems):
    sid = lax.axis_index("s")
    wid = sid * NC + lax.axis_index("c")
    t0 = wid * TPW

    idx_descs = [
        pltpu.async_copy(
            ids_hbm.at[b, pl.ds(t0 + g * CS, CS)],
            idx_v.at[g, pl.ds(b * CS, CS)],
            isem,
        )
        for g in range(NGRP)
        for b in range(B)
    ]
    for d in idx_descs:
        d.wait()

    def start_gather(g):
        p = g % NBUF
        return pltpu.async_copy(
            wte_hbm.at[idx_v.at[g]], rows_v[p], gsems[p]
        )

    def start_wpe(g):
        p = g % NBUF
        return pltpu.async_copy(
            wpe_hbm.at[pl.ds(t0 + g * CS, CS), :], wpe_v[p], wsems[p]
        )

    def start_stores(g):
        p = g % NBUF
        return [
            pltpu.async_copy(
                rows_v[p].at[pl.ds(b * CS, CS)],
                out_hbm.at[b, pl.ds(t0 + g * CS, CS), :],
                ssems[p],
            )
            for b in range(B)
        ]

    def add_wpe(g):
        p = g % NBUF
        buf = rows_v[p]
        wp = wpe_v[p]

        @plsc.parallel_loop(0, CS, unroll=1)
        def _(i):
            for j in range(D // L):
                sl = pl.ds(j * L, L)
                x = wp[i, sl]
                for b in range(B):
                    plsc.addupdate(buf.at[b * CS + i, sl], x)

    g_descs = {}
    w_descs = {}
    s_descs = {}
    for g in range(NBUF):
        g_descs[g] = start_gather(g)
        w_descs[g] = start_wpe(g)
    for g in range(NGRP):
        j = g + NBUF - 1
        if 1 <= g and j < NGRP:
            for d in s_descs[g - 1]:
                d.wait()
            g_descs[j] = start_gather(j)
            w_descs[j] = start_wpe(j)
        g_descs[g].wait()
        w_descs[g].wait()
        add_wpe(g)
        s_descs[g] = start_stores(g)
    for g in range(max(0, NGRP - NBUF), NGRP):
        for d in s_descs[g]:
            d.wait()


def kernel(input_ids, wte, wpe):
    ids32 = input_ids.astype(jnp.int32)
    return _emb_lookup(ids32, wte, wpe)


# speedup vs baseline: 1.0299x; 1.0026x over previous
import functools

import jax
import jax.numpy as jnp
from jax import lax
from jax.experimental import pallas as pl
from jax.experimental.pallas import tpu as pltpu
from jax.experimental.pallas import tpu_sc as plsc

B, T, D = 4, 2048, 768
VOCAB = 50257
NC, NS, L = 2, 16, 16
NW = NC * NS
TPW = T // NW
CS = 16
NGRP = TPW // CS
NBUF = 2

_mesh = plsc.VectorSubcoreMesh(
    core_axis_name="c", subcore_axis_name="s", num_cores=NC, num_subcores=NS
)


@functools.partial(
    pl.kernel,
    out_type=jax.ShapeDtypeStruct((B, T, D), jnp.float32),
    mesh=_mesh,
    scratch_types=[
        pltpu.VMEM((NGRP, B * CS), jnp.int32),
        [pltpu.VMEM((CS, D), jnp.float32) for _ in range(NBUF)],
        [pltpu.VMEM((B * CS, D), jnp.float32) for _ in range(NBUF)],
        [pltpu.SemaphoreType.DMA for _ in range(NGRP)],
        [pltpu.SemaphoreType.DMA for _ in range(NBUF)],
        [pltpu.SemaphoreType.DMA for _ in range(NBUF)],
        [pltpu.SemaphoreType.DMA for _ in range(NBUF)],
    ],
)
def _emb_lookup(ids_hbm, wte_hbm, wpe_hbm, out_hbm,
                idx_v, wpe_v, rows_v, isems, wsems, gsems, ssems):
    sid = lax.axis_index("s")
    wid = sid * NC + lax.axis_index("c")
    t0 = wid * TPW

    idx_descs = {
        g: [
            pltpu.async_copy(
                ids_hbm.at[b, pl.ds(t0 + g * CS, CS)],
                idx_v.at[g, pl.ds(b * CS, CS)],
                isems[g],
            )
            for b in range(B)
        ]
        for g in range(NGRP)
    }

    def start_gather(g):
        p = g % NBUF
        for d in idx_descs[g]:
            d.wait()
        return pltpu.async_copy(
            wte_hbm.at[idx_v.at[g]], rows_v[p], gsems[p]
        )

    def start_wpe(g):
        p = g % NBUF
        return pltpu.async_copy(
            wpe_hbm.at[pl.ds(t0 + g * CS, CS), :], wpe_v[p], wsems[p]
        )

    def start_stores(g):
        p = g % NBUF
        return [
            pltpu.async_copy(
                rows_v[p].at[pl.ds(b * CS, CS)],
                out_hbm.at[b, pl.ds(t0 + g * CS, CS), :],
                ssems[p],
            )
            for b in range(B)
        ]

    def add_wpe(g):
        p = g % NBUF
        buf = rows_v[p]
        wp = wpe_v[p]

        @plsc.parallel_loop(0, CS, unroll=1)
        def _(i):
            for j in range(D // L):
                sl = pl.ds(j * L, L)
                x = wp[i, sl]
                for b in range(B):
                    plsc.addupdate(buf.at[b * CS + i, sl], x)

    g_descs = {}
    w_descs = {}
    s_descs = {}
    for g in range(NBUF):
        g_descs[g] = start_gather(g)
        w_descs[g] = start_wpe(g)
    for g in range(NGRP):
        j = g + NBUF - 1
        if 1 <= g and j < NGRP:
            for d in s_descs[g - 1]:
                d.wait()
            g_descs[j] = start_gather(j)
            w_descs[j] = start_wpe(j)
        g_descs[g].wait()
        w_descs[g].wait()
        add_wpe(g)
        s_descs[g] = start_stores(g)
    for g in range(max(0, NGRP - NBUF), NGRP):
        for d in s_descs[g]:
            d.wait()


def kernel(input_ids, wte, wpe):
    ids32 = input_ids.astype(jnp.int32)
    return _emb_lookup(ids32, wte, wpe)
